# baseline (speedup 1.0000x reference)
import functools

import jax
import jax.numpy as jnp
from jax import lax
from jax.experimental import pallas as pl
from jax.experimental.pallas import tpu as pltpu
from jax.experimental.pallas import tpu_sc as plsc

_N = 10000
_E = 160000
_HID = 16
_B = 8
_WCOL = 128
_NBLK = 400
_SC_NC = 2
_SC_NS = 16
_GE = 80
_QB = 32
_STRIPE = _N // _SC_NS
_PREC = jax.lax.Precision.DEFAULT
_EXACT = jax.lax.Precision.HIGHEST
_F32 = jnp.float32


def _stripe_plan(sid, fn):
    s0 = 640
    last = _N - (_SC_NS - 1) * s0
    r0 = pl.multiple_of(sid * s0, 8)

    @pl.when(sid < _SC_NS - 1)
    def _():
        fn(r0, s0)

    @pl.when(sid == _SC_NS - 1)
    def _():
        fn((_SC_NS - 1) * s0, last)


@functools.lru_cache(maxsize=None)
def _make_sc_scatter(nchunk, width, nit):
    assert nit % _QB == 0
    nq = nit // _QB
    mesh = plsc.VectorSubcoreMesh(core_axis_name="c", subcore_axis_name="s")
    out_type = jax.ShapeDtypeStruct((nchunk, _N, width), _F32)
    scratch = [
        pltpu.VMEM((_QB, _GE), jnp.int32),
        pltpu.VMEM((_QB, _GE), jnp.int32),
        pltpu.VMEM((_GE, width), _F32),
        pltpu.VMEM((_GE, width), _F32),
        pltpu.VMEM_SHARED((_N + 8, width), _F32),
        pltpu.SemaphoreType.DMA,
        pltpu.SemaphoreType.DMA,
        pltpu.SemaphoreType.DMA,
        pltpu.SemaphoreType.DMA,
    ]

    @functools.partial(pl.kernel, out_type=out_type, mesh=mesh,
                       scratch_types=scratch)
    def sck(src_hbm, dst_hbm, zeros_hbm, *rest):
        y_refs = rest[:nchunk]
        s_ref = rest[nchunk]
        (sidx, didx, rows_a, rows_b, acc,
         gsem_a, gsem_b, ssem_a, ssem_b) = rest[nchunk + 1:]
        cid = lax.axis_index("c")
        sid = lax.axis_index("s")

        def _gath(yr, b, rows, sem):
            return pltpu.make_async_copy(yr.at[sidx.at[b]], rows, sem)

        def _scat(b, rows, sem):
            return pltpu.make_async_copy(rows, acc.at[didx.at[b]], sem)

        def _run_quarter(yr, q):
            pltpu.sync_copy(src_hbm.at[sid, pl.ds(q * _QB, _QB)], sidx)
            pltpu.sync_copy(dst_hbm.at[sid, pl.ds(q * _QB, _QB)], didx)
            _gath(yr, 0, rows_a, gsem_a).start()

            def body(t, carry):
                b0 = 2 * t
                b1 = b0 + 1
                _gath(yr, b0, rows_a, gsem_a).wait()

                @pl.when(t >= 1)
                def _():
                    _scat(b0, rows_b, ssem_b).wait()

                _gath(yr, b1, rows_b, gsem_b).start()
                _scat(b0, rows_a, ssem_a).start(add=True)
                _gath(yr, b1, rows_b, gsem_b).wait()

                @pl.when(t < _QB // 2 - 1)
                def _():
                    _scat(b1, rows_a, ssem_a).wait()
                    _gath(yr, b0 + 2, rows_a, gsem_a).start()

                _scat(b1, rows_b, ssem_b).start(add=True)
                return carry

            lax.fori_loop(0, _QB // 2, body, 0)
            _scat(0, rows_a, ssem_a).wait()
            _scat(0, rows_b, ssem_b).wait()

        for c in range(nchunk):
            @pl.when(cid == (c % _SC_NC))
            def _(yr=y_refs[c], c=c):
                _stripe_plan(sid, lambda a, m: pltpu.sync_copy(
                    zeros_hbm.at[pl.ds(a, m)], acc.at[pl.ds(a, m)]))
                plsc.subcore_barrier()
                for q in range(nq):
                    _run_quarter(yr, q)
                plsc.subcore_barrier()
                _stripe_plan(sid, lambda a, m: pltpu.sync_copy(
                    acc.at[pl.ds(a, m)], s_ref.at[c, pl.ds(a, m)]))
        return None

    return sck


@functools.lru_cache(maxsize=None)
def _make_sc_deg(nit):
    assert nit % _QB == 0
    nq = nit // _QB
    mesh = plsc.VectorSubcoreMesh(core_axis_name="c", subcore_axis_name="s")
    out_type = jax.ShapeDtypeStruct((_SC_NC, _N, _WCOL), _F32)
    scratch = [
        pltpu.VMEM((_QB, _GE), jnp.int32),
        pltpu.VMEM((_GE, _WCOL), _F32),
        pltpu.VMEM_SHARED((_N + 8, _WCOL), _F32),
        pltpu.SemaphoreType.DMA,
        pltpu.SemaphoreType.DMA,
    ]

    @functools.partial(pl.kernel, out_type=out_type, mesh=mesh,
                       scratch_types=scratch)
    def dk(dst_hbm, zeros_hbm, ones_hbm, s_ref, didx, ones_v, acc,
           ssem_a, ssem_b):
        cid = lax.axis_index("c")
        sid = lax.axis_index("s")
        w = cid * _SC_NS + sid

        def _scat(b, sem):
            return pltpu.make_async_copy(ones_v, acc.at[didx.at[b]], sem)

        pltpu.sync_copy(ones_hbm, ones_v)
        _stripe_plan(sid, lambda a, m: pltpu.sync_copy(
            zeros_hbm.at[pl.ds(a, m)], acc.at[pl.ds(a, m)]))
        plsc.subcore_barrier()
        for q in range(nq):
            pltpu.sync_copy(dst_hbm.at[w, pl.ds(q * _QB, _QB)], didx)

            def body(t, carry):
                b0 = 2 * t

                @pl.when(t >= 1)
                def _():
                    _scat(b0, ssem_a).wait()
                    _scat(b0, ssem_b).wait()

                _scat(b0, ssem_a).start(add=True)
                _scat(b0 + 1, ssem_b).start(add=True)
                return carry

            lax.fori_loop(0, _QB // 2, body, 0)
            _scat(0, ssem_a).wait()
            _scat(0, ssem_b).wait()
        plsc.subcore_barrier()
        _stripe_plan(sid, lambda a, m: pltpu.sync_copy(
            acc.at[pl.ds(a, m)], s_ref.at[cid, pl.ds(a, m)]))
        return None

    return dk


def _pad_edges(idx, fill):
    epad = _SC_NS * ((-(-_E // (_SC_NS * _GE * _QB))) * _QB) * _GE - _E
    if epad:
        pad_vals = _N + (jnp.arange(epad, dtype=jnp.int32) % 8) \
            if fill is None else jnp.zeros((epad,), jnp.int32)
        idx = jnp.concatenate([idx, pad_vals])
    return idx


def _sc_scatter_call(nchunk, width, n_edges, src, dst, y_list):
    del n_edges
    srcp = _pad_edges(src, fill=0)
    dstp = _pad_edges(dst, fill=None)
    nit = srcp.shape[0] // (_SC_NS * _GE)
    src3 = srcp.reshape(_SC_NS, nit, _GE)
    dst3 = dstp.reshape(_SC_NS, nit, _GE)
    zeros = jnp.zeros((_N, width), _F32)
    fn = _make_sc_scatter(nchunk, width, nit)
    return fn(src3, dst3, zeros, *y_list)


def _sc_deg_call(src):
    dstp = _pad_edges(src, fill=None)
    nw = _SC_NC * _SC_NS
    nit = dstp.shape[0] // (nw * _GE)
    dst3 = dstp.reshape(nw, nit, _GE)
    zeros = jnp.zeros((_N, _WCOL), _F32)
    ones = jnp.ones((_GE, _WCOL), _F32)
    d3 = _make_sc_deg(nit)(dst3, zeros, ones)
    return (d3[0, :, 0] + d3[1, :, 0])[:, None]


def _gated(x, m1, m2, m3, b1, b2, b3):
    n, cin = x.shape
    cout = m1.shape[1]

    def body(xr, m1r, m2r, m3r, b1r, b2r, b3r, outr):
        xb = xr[...]
        p = jnp.dot(xb, m1r[...], precision=_PREC) + b1r[...]
        q = jax.nn.sigmoid(jnp.dot(xb, m2r[...], precision=_PREC) + b2r[...])
        r = jnp.dot(xb, m3r[...], precision=_PREC) + b3r[...]
        outr[...] = jax.nn.relu(p * q + r)

    wspec = pl.BlockSpec((cin, cout), lambda i: (0, 0))
    bspec = pl.BlockSpec((1, cout), lambda i: (0, 0))
    return pl.pallas_call(
        body,
        grid=(n // _NBLK,),
        in_specs=[pl.BlockSpec((_NBLK, cin), lambda i: (i, 0)),
                  wspec, wspec, wspec, bspec, bspec, bspec],
        out_specs=pl.BlockSpec((_NBLK, cout), lambda i: (i, 0)),
        out_shape=jax.ShapeDtypeStruct((n, cout), _F32),
    )(x, m1, m2, m3, b1, b2, b3)


def _dinv_of(d):
    return jnp.where(d > 0, 1.0 / jnp.sqrt(jnp.maximum(d, 1.0)), 0.0)


def _cheb_first(t0, deg, w0bd, nchunk):
    n, c = t0.shape

    def body(t0r, degr, w0r, oar, y3r):
        dinv = _dinv_of(degr[...])
        tb = t0r[...]
        oar[...] = jnp.dot(tb, w0r[...], precision=_PREC)
        y3r[...] = (tb * dinv)[None]

    return pl.pallas_call(
        body,
        grid=(n // _NBLK, nchunk),
        in_specs=[pl.BlockSpec((_NBLK, _WCOL), lambda i, c: (i, c)),
                  pl.BlockSpec((_NBLK, 1), lambda i, c: (i, 0)),
                  pl.BlockSpec((_WCOL, _WCOL), lambda i, c: (0, 0))],
        out_specs=[pl.BlockSpec((_NBLK, _WCOL), lambda i, c: (i, c)),
                   pl.BlockSpec((1, _NBLK, _WCOL), lambda i, c: (c, i, 0))],
        out_shape=[jax.ShapeDtypeStruct((n, c), _F32),
                   jax.ShapeDtypeStruct((nchunk, n, _WCOL), _F32)],
    )(t0, deg, w0bd)


def _cheb_step1(s3, deg, wbd, oa, nchunk, n, c):
    def body(s3r, degr, wr, oir, oar, txr, y3r):
        dinv = _dinv_of(degr[...])
        tx = -dinv * s3r[0]
        txr[...] = tx
        oar[...] = oir[...] + jnp.dot(tx, wr[...], precision=_PREC)
        y3r[...] = (tx * dinv)[None]

    return pl.pallas_call(
        body,
        grid=(n // _NBLK, nchunk),
        in_specs=[pl.BlockSpec((1, _NBLK, _WCOL), lambda i, c: (c, i, 0)),
                  pl.BlockSpec((_NBLK, 1), lambda i, c: (i, 0)),
                  pl.BlockSpec((_WCOL, _WCOL), lambda i, c: (0, 0)),
                  pl.BlockSpec((_NBLK, _WCOL), lambda i, c: (i, c))],
        out_specs=[pl.BlockSpec((_NBLK, _WCOL), lambda i, c: (i, c)),
                   pl.BlockSpec((_NBLK, _WCOL), lambda i, c: (i, c)),
                   pl.BlockSpec((1, _NBLK, _WCOL), lambda i, c: (c, i, 0))],
        out_shape=[jax.ShapeDtypeStruct((n, c), _F32),
                   jax.ShapeDtypeStruct((n, c), _F32),
                   jax.ShapeDtypeStruct((nchunk, n, _WCOL), _F32)],
        input_output_aliases={3: 0},
    )(s3, deg, wbd, oa)


def _cheb_step2(s3, deg, wbd, txm2, oa, nchunk, n, c):
    def body(s3r, degr, wr, tmr, oir, oar, txr, y3r):
        dinv = _dinv_of(degr[...])
        tx = -2.0 * dinv * s3r[0] - tmr[...]
        txr[...] = tx
        oar[...] = oir[...] + jnp.dot(tx, wr[...], precision=_PREC)
        y3r[...] = (tx * dinv)[None]

    return pl.pallas_call(
        body,
        grid=(n // _NBLK, nchunk),
        in_specs=[pl.BlockSpec((1, _NBLK, _WCOL), lambda i, c: (c, i, 0)),
                  pl.BlockSpec((_NBLK, 1), lambda i, c: (i, 0)),
                  pl.BlockSpec((_WCOL, _WCOL), lambda i, c: (0, 0)),
                  pl.BlockSpec((_NBLK, _WCOL), lambda i, c: (i, c)),
                  pl.BlockSpec((_NBLK, _WCOL), lambda i, c: (i, c))],
        out_specs=[pl.BlockSpec((_NBLK, _WCOL), lambda i, c: (i, c)),
                   pl.BlockSpec((_NBLK, _WCOL), lambda i, c: (i, c)),
                   pl.BlockSpec((1, _NBLK, _WCOL), lambda i, c: (c, i, 0))],
        out_shape=[jax.ShapeDtypeStruct((n, c), _F32),
                   jax.ShapeDtypeStruct((n, c), _F32),
                   jax.ShapeDtypeStruct((nchunk, n, _WCOL), _F32)],
        input_output_aliases={4: 0},
    )(s3, deg, wbd, txm2, oa)


def _cheb_last(s3, deg, wbd, txm2, brow, oa, nchunk, n, c):
    def body(s3r, degr, wr, tmr, br, oir, outr):
        dinv = _dinv_of(degr[...])
        tx = -2.0 * dinv * s3r[0] - tmr[...]
        outr[...] = jax.nn.relu(
            oir[...] + jnp.dot(tx, wr[...], precision=_PREC) + br[...])

    return pl.pallas_call(
        body,
        grid=(n // _NBLK, nchunk),
        in_specs=[pl.BlockSpec((1, _NBLK, _WCOL), lambda i, c: (c, i, 0)),
                  pl.BlockSpec((_NBLK, 1), lambda i, c: (i, 0)),
                  pl.BlockSpec((_WCOL, _WCOL), lambda i, c: (0, 0)),
                  pl.BlockSpec((_NBLK, _WCOL), lambda i, c: (i, c)),
                  pl.BlockSpec((1, _WCOL), lambda i, c: (0, c)),
                  pl.BlockSpec((_NBLK, _WCOL), lambda i, c: (i, c))],
        out_specs=pl.BlockSpec((_NBLK, _WCOL), lambda i, c: (i, c)),
        out_shape=jax.ShapeDtypeStruct((n, c), _F32),
        input_output_aliases={5: 0},
    )(s3, deg, wbd, txm2, brow, oa)


def _bn_relu(t2, g, b):
    n, tt = t2.shape

    def body(tr, gr, br, outr):
        tb = tr[...]
        m = jnp.mean(tb, axis=1, keepdims=True)
        v = jnp.mean((tb - m) ** 2, axis=1, keepdims=True)
        outr[...] = jax.nn.relu((tb - m) / jnp.sqrt(v + 1e-5) * gr[...] + br[...])

    return pl.pallas_call(
        body,
        grid=(n // _NBLK,),
        in_specs=[pl.BlockSpec((_NBLK, tt), lambda i: (i, 0)),
                  pl.BlockSpec((_NBLK, 1), lambda i: (i, 0)),
                  pl.BlockSpec((_NBLK, 1), lambda i: (i, 0))],
        out_specs=pl.BlockSpec((_NBLK, tt), lambda i: (i, 0)),
        out_shape=jax.ShapeDtypeStruct((n, tt), _F32),
    )(t2, g, b)


def _final_linear(lw, h, lb):
    n = lw.shape[0]
    bb = h.shape[1]
    rows = 200

    def body(lwr, hr, lbr, outr):
        outr[...] = jnp.dot(lwr[...], hr[...], precision=_PREC) + lbr[...]

    return pl.pallas_call(
        body,
        grid=(n // rows,),
        in_specs=[pl.BlockSpec((rows, n), lambda i: (i, 0)),
                  pl.BlockSpec((n, bb), lambda i: (0, 0)),
                  pl.BlockSpec((rows, 1), lambda i: (i, 0))],
        out_specs=pl.BlockSpec((rows, bb), lambda i: (i, 0)),
        out_shape=jax.ShapeDtypeStruct((n, bb), _F32),
    )(lw, h, lb)


def _tconv_in_mat(w, t_in):
    t_out = t_in - (w.shape[-1] - 1)
    wr = w[:, 0, :].T
    es = jnp.stack([jnp.eye(t_out, t_in, k) for k in range(w.shape[-1])])
    mblk = jnp.einsum('ktn,ko->nto', es, wr, precision=_EXACT).reshape(t_in, t_out * _HID)
    return jnp.kron(jnp.eye(_B), mblk), t_out


def _tconv_out_mat(w, t_in):
    t_out = t_in - (w.shape[-1] - 1)
    wr = w[0].T
    es = jnp.stack([jnp.eye(t_out, t_in, k) for k in range(w.shape[-1])])
    mblk = jnp.einsum('ktn,kc->nct', es, wr, precision=_EXACT).reshape(t_in * _HID, t_out)
    return jnp.kron(jnp.eye(_B), mblk), t_out


def _bdiag(w, reps):
    return jnp.kron(jnp.eye(reps), w)


def _stconv(xin, src, dst, deg, t_in, p):
    m1, t1 = _tconv_in_mat(p['t1w1'], t_in)
    m2, _ = _tconv_in_mat(p['t1w2'], t_in)
    m3, _ = _tconv_in_mat(p['t1w3'], t_in)
    b1 = jnp.tile(p['t1b1'], _B * t1)[None]
    b2 = jnp.tile(p['t1b2'], _B * t1)[None]
    b3 = jnp.tile(p['t1b3'], _B * t1)[None]
    t0 = _gated(xin, m1, m2, m3, b1, b2, b3)

    cc = _B * t1 * _HID
    nchunk = cc // _WCOL
    assert cc % _WCOL == 0
    reps = _WCOL // _HID
    chw = p['chw']
    oa, y3 = _cheb_first(t0, deg, _bdiag(chw[0], reps), nchunk)
    y_list = [y3[c] for c in range(nchunk)]
    s3 = _sc_scatter_call(nchunk, _WCOL, _E, src, dst, y_list)
    oa, tx1, y3 = _cheb_step1(s3, deg, _bdiag(chw[1], reps), oa, nchunk, _N, cc)
    txm2 = t0
    txprev = tx1
    for k in range(2, chw.shape[0] - 1):
        y_list = [y3[c] for c in range(nchunk)]
        s3 = _sc_scatter_call(nchunk, _WCOL, _E, src, dst, y_list)
        oa, txk, y3 = _cheb_step2(s3, deg, _bdiag(chw[k], reps), txm2, oa,
                                  nchunk, _N, cc)
        txm2, txprev = txprev, txk
    y_list = [y3[c] for c in range(nchunk)]
    s3 = _sc_scatter_call(nchunk, _WCOL, _E, src, dst, y_list)
    brow = jnp.tile(p['chb'], _B * t1)[None]
    g = _cheb_last(s3, deg, _bdiag(chw[-1], reps), txm2, brow, oa,
                   nchunk, _N, cc)

    m1o, t2 = _tconv_out_mat(p['t2w1'], t1)
    m2o, _ = _tconv_out_mat(p['t2w2'], t1)
    m3o, _ = _tconv_out_mat(p['t2w3'], t1)
    b1o = jnp.tile(p['t2b1'], _B * t2)[None]
    b2o = jnp.tile(p['t2b2'], _B * t2)[None]
    b3o = jnp.tile(p['t2b3'], _B * t2)[None]
    t2a = _gated(g, m1o, m2o, m3o, b1o, b2o, b3o)

    return _bn_relu(t2a, p['bng'][:, None], p['bnb'][:, None])


def kernel(x, edge_index, a_t1w1, a_t1b1, a_t2w1, a_t2b1, a_t1w2, a_t1b2,
           a_t2w2, a_t2b2, a_t1w3, a_t1b3, a_t2w3, a_t2b3, a_chw, a_chb,
           a_bng, a_bnb, b_t1w1, b_t1b1, b_t2w1, b_t2b1, b_t1w2, b_t1b2,
           b_t2w2, b_t2b2, b_t1w3, b_t1b3, b_t2w3, b_t2b3, b_chw, b_chb,
           b_bng, b_bnb, lin_w, lin_b):
    src = edge_index[0]
    dst = edge_index[1]

    x_nm = jnp.transpose(x[:, :, :, 0], (2, 0, 1)).reshape(_N, -1)

    deg = _sc_deg_call(src)

    pa = {'t1w1': a_t1w1, 't1b1': a_t1b1, 't1w2': a_t1w2, 't1b2': a_t1b2,
          't1w3': a_t1w3, 't1b3': a_t1b3, 't2w1': a_t2w1, 't2b1': a_t2b1,
          't2w2': a_t2w2, 't2b2': a_t2b2, 't2w3': a_t2w3, 't2b3': a_t2b3,
          'chw': a_chw, 'chb': a_chb, 'bng': a_bng, 'bnb': a_bnb}
    pb = {'t1w1': b_t1w1, 't1b1': b_t1b1, 't1w2': b_t1w2, 't1b2': b_t1b2,
          't1w3': b_t1w3, 't1b3': b_t1b3, 't2w1': b_t2w1, 't2b1': b_t2b1,
          't2w2': b_t2w2, 't2b2': b_t2b2, 't2w3': b_t2w3, 't2b3': b_t2b3,
          'chw': b_chw, 'chb': b_chb, 'bng': b_bng, 'bnb': b_bnb}

    h = _stconv(x_nm, src, dst, deg, 12, pa)
    h = _stconv(h, src, dst, deg, 8, pb)

    hsel = h.reshape(_N, _B, 4)[:, :, 3]
    out_nm = _final_linear(lin_w, hsel, lin_b[:, None])
    return out_nm.T


# speedup vs baseline: 1.1193x; 1.1193x over previous
import functools

import jax
import jax.numpy as jnp
from jax import lax
from jax.experimental import pallas as pl
from jax.experimental.pallas import tpu as pltpu
from jax.experimental.pallas import tpu_sc as plsc

_N = 10000
_E = 160000
_HID = 16
_B = 8
_WCOL = 128
_NBLK = 400
_SC_NC = 2
_SC_NS = 16
_GE = 80
_QB = 32
_STRIPE = _N // _SC_NS
_PREC = jax.lax.Precision.DEFAULT
_EXACT = jax.lax.Precision.HIGHEST
_F32 = jnp.float32


def _stripe_plan(sid, fn):
    s0 = 640
    last = _N - (_SC_NS - 1) * s0
    r0 = pl.multiple_of(sid * s0, 8)

    @pl.when(sid < _SC_NS - 1)
    def _():
        fn(r0, s0)

    @pl.when(sid == _SC_NS - 1)
    def _():
        fn((_SC_NS - 1) * s0, last)


@functools.lru_cache(maxsize=None)
def _make_sc_scatter(nchunk, width, nit):
    assert nit % _QB == 0
    nq = nit // _QB
    mesh = plsc.VectorSubcoreMesh(core_axis_name="c", subcore_axis_name="s")
    out_type = jax.ShapeDtypeStruct((nchunk, _N, width), _F32)
    scratch = [
        pltpu.VMEM((_QB, _GE), jnp.int32),
        pltpu.VMEM((_QB, _GE), jnp.int32),
        pltpu.VMEM((_GE, width), _F32),
        pltpu.VMEM((_GE, width), _F32),
        pltpu.VMEM_SHARED((_N + 8, width), _F32),
        pltpu.SemaphoreType.DMA,
        pltpu.SemaphoreType.DMA,
        pltpu.SemaphoreType.DMA,
        pltpu.SemaphoreType.DMA,
    ]

    @functools.partial(pl.kernel, out_type=out_type, mesh=mesh,
                       scratch_types=scratch)
    def sck(src_hbm, dst_hbm, zeros_hbm, *rest):
        y_refs = rest[:nchunk]
        s_ref = rest[nchunk]
        (sidx, didx, rows_a, rows_b, acc,
         gsem_a, gsem_b, ssem_a, ssem_b) = rest[nchunk + 1:]
        cid = lax.axis_index("c")
        sid = lax.axis_index("s")

        def _gath(yr, b, rows, sem):
            return pltpu.make_async_copy(yr.at[sidx.at[b]], rows, sem)

        def _scat(b, rows, sem):
            return pltpu.make_async_copy(rows, acc.at[didx.at[b]], sem)

        def _run_quarter(yr, q):
            pltpu.sync_copy(src_hbm.at[sid, pl.ds(q * _QB, _QB)], sidx)
            pltpu.sync_copy(dst_hbm.at[sid, pl.ds(q * _QB, _QB)], didx)
            _gath(yr, 0, rows_a, gsem_a).start()

            def body(t, carry):
                b0 = 2 * t
                b1 = b0 + 1
                _gath(yr, b0, rows_a, gsem_a).wait()

                @pl.when(t >= 1)
                def _():
                    _scat(b0, rows_b, ssem_b).wait()

                _gath(yr, b1, rows_b, gsem_b).start()
                _scat(b0, rows_a, ssem_a).start(add=True)
                _gath(yr, b1, rows_b, gsem_b).wait()

                @pl.when(t < _QB // 2 - 1)
                def _():
                    _scat(b1, rows_a, ssem_a).wait()
                    _gath(yr, b0 + 2, rows_a, gsem_a).start()

                _scat(b1, rows_b, ssem_b).start(add=True)
                return carry

            lax.fori_loop(0, _QB // 2, body, 0)
            _scat(0, rows_a, ssem_a).wait()
            _scat(0, rows_b, ssem_b).wait()

        for c in range(nchunk):
            @pl.when(cid == (c % _SC_NC))
            def _(yr=y_refs[c], c=c):
                _stripe_plan(sid, lambda a, m: pltpu.sync_copy(
                    zeros_hbm.at[pl.ds(a, m)], acc.at[pl.ds(a, m)]))
                plsc.subcore_barrier()
                for q in range(nq):
                    _run_quarter(yr, q)
                plsc.subcore_barrier()
                _stripe_plan(sid, lambda a, m: pltpu.sync_copy(
                    acc.at[pl.ds(a, m)], s_ref.at[c, pl.ds(a, m)]))
        return None

    return sck


@functools.lru_cache(maxsize=None)
def _make_sc_deg(nit):
    assert nit % _QB == 0
    nq = nit // _QB
    mesh = plsc.VectorSubcoreMesh(core_axis_name="c", subcore_axis_name="s")
    out_type = jax.ShapeDtypeStruct((_SC_NC, _N, _WCOL), _F32)
    scratch = [
        pltpu.VMEM((_QB, _GE), jnp.int32),
        pltpu.VMEM((_GE, _WCOL), _F32),
        pltpu.VMEM_SHARED((_N + 8, _WCOL), _F32),
        pltpu.SemaphoreType.DMA,
        pltpu.SemaphoreType.DMA,
    ]

    @functools.partial(pl.kernel, out_type=out_type, mesh=mesh,
                       scratch_types=scratch)
    def dk(dst_hbm, zeros_hbm, ones_hbm, s_ref, didx, ones_v, acc,
           ssem_a, ssem_b):
        cid = lax.axis_index("c")
        sid = lax.axis_index("s")
        w = cid * _SC_NS + sid

        def _scat(b, sem):
            return pltpu.make_async_copy(ones_v, acc.at[didx.at[b]], sem)

        pltpu.sync_copy(ones_hbm, ones_v)
        _stripe_plan(sid, lambda a, m: pltpu.sync_copy(
            zeros_hbm.at[pl.ds(a, m)], acc.at[pl.ds(a, m)]))
        plsc.subcore_barrier()
        for q in range(nq):
            pltpu.sync_copy(dst_hbm.at[w, pl.ds(q * _QB, _QB)], didx)

            def body(t, carry):
                b0 = 2 * t

                @pl.when(t >= 1)
                def _():
                    _scat(b0, ssem_a).wait()
                    _scat(b0, ssem_b).wait()

                _scat(b0, ssem_a).start(add=True)
                _scat(b0 + 1, ssem_b).start(add=True)
                return carry

            lax.fori_loop(0, _QB // 2, body, 0)
            _scat(0, ssem_a).wait()
            _scat(0, ssem_b).wait()
        plsc.subcore_barrier()
        _stripe_plan(sid, lambda a, m: pltpu.sync_copy(
            acc.at[pl.ds(a, m)], s_ref.at[cid, pl.ds(a, m)]))
        return None

    return dk


def _pad_slabs(idx, nslabs, dummy):
    per = idx.shape[0] // nslabs
    nit = -(-(per // _GE) // _QB) * _QB
    padn = nit * _GE - per
    x = idx.reshape(nslabs, per)
    if padn:
        if dummy:
            pad = _N + (jnp.zeros((nslabs, padn), jnp.int32)
                        + jnp.arange(padn, dtype=jnp.int32) % 8)
        else:
            pad = jnp.zeros((nslabs, padn), jnp.int32)
        x = jnp.concatenate([x, pad], axis=1)
    return x.reshape(nslabs, nit, _GE)


def _sc_scatter_call(nchunk, width, n_edges, src, dst, y_list):
    del n_edges
    src3 = _pad_slabs(src, _SC_NS, dummy=False)
    dst3 = _pad_slabs(dst, _SC_NS, dummy=True)
    zeros = jnp.zeros((_N, width), _F32)
    fn = _make_sc_scatter(nchunk, width, dst3.shape[1])
    return fn(src3, dst3, zeros, *y_list)


def _sc_deg_call(src):
    dst3 = _pad_slabs(src, _SC_NC * _SC_NS, dummy=True)
    zeros = jnp.zeros((_N, _WCOL), _F32)
    ones = jnp.ones((_GE, _WCOL), _F32)
    d3 = _make_sc_deg(dst3.shape[1])(dst3, zeros, ones)
    return (d3[0, :, 0] + d3[1, :, 0])[:, None]


def _gated(x, m1, m2, m3, b1, b2, b3):
    n, cin = x.shape
    cout = m1.shape[1]

    def body(xr, m1r, m2r, m3r, b1r, b2r, b3r, outr):
        xb = xr[...]
        p = jnp.dot(xb, m1r[...], precision=_PREC) + b1r[...]
        q = jax.nn.sigmoid(jnp.dot(xb, m2r[...], precision=_PREC) + b2r[...])
        r = jnp.dot(xb, m3r[...], precision=_PREC) + b3r[...]
        outr[...] = jax.nn.relu(p * q + r)

    wspec = pl.BlockSpec((cin, cout), lambda i: (0, 0))
    bspec = pl.BlockSpec((1, cout), lambda i: (0, 0))
    return pl.pallas_call(
        body,
        grid=(n // _NBLK,),
        in_specs=[pl.BlockSpec((_NBLK, cin), lambda i: (i, 0)),
                  wspec, wspec, wspec, bspec, bspec, bspec],
        out_specs=pl.BlockSpec((_NBLK, cout), lambda i: (i, 0)),
        out_shape=jax.ShapeDtypeStruct((n, cout), _F32),
    )(x, m1, m2, m3, b1, b2, b3)


def _dinv_of(d):
    return jnp.where(d > 0, 1.0 / jnp.sqrt(jnp.maximum(d, 1.0)), 0.0)


def _cheb_first(t0, deg, w0bd, nchunk):
    n, c = t0.shape

    def body(t0r, degr, w0r, oar, y3r):
        dinv = _dinv_of(degr[...])
        tb = t0r[...]
        oar[...] = jnp.dot(tb, w0r[...], precision=_PREC)
        y3r[...] = (tb * dinv)[None]

    return pl.pallas_call(
        body,
        grid=(n // _NBLK, nchunk),
        in_specs=[pl.BlockSpec((_NBLK, _WCOL), lambda i, c: (i, c)),
                  pl.BlockSpec((_NBLK, 1), lambda i, c: (i, 0)),
                  pl.BlockSpec((_WCOL, _WCOL), lambda i, c: (0, 0))],
        out_specs=[pl.BlockSpec((_NBLK, _WCOL), lambda i, c: (i, c)),
                   pl.BlockSpec((1, _NBLK, _WCOL), lambda i, c: (c, i, 0))],
        out_shape=[jax.ShapeDtypeStruct((n, c), _F32),
                   jax.ShapeDtypeStruct((nchunk, n, _WCOL), _F32)],
    )(t0, deg, w0bd)


def _cheb_step1(s3, deg, wbd, oa, nchunk, n, c):
    def body(s3r, degr, wr, oir, oar, txr, y3r):
        dinv = _dinv_of(degr[...])
        tx = -dinv * s3r[0]
        txr[...] = tx
        oar[...] = oir[...] + jnp.dot(tx, wr[...], precision=_PREC)
        y3r[...] = (tx * dinv)[None]

    return pl.pallas_call(
        body,
        grid=(n // _NBLK, nchunk),
        in_specs=[pl.BlockSpec((1, _NBLK, _WCOL), lambda i, c: (c, i, 0)),
                  pl.BlockSpec((_NBLK, 1), lambda i, c: (i, 0)),
                  pl.BlockSpec((_WCOL, _WCOL), lambda i, c: (0, 0)),
                  pl.BlockSpec((_NBLK, _WCOL), lambda i, c: (i, c))],
        out_specs=[pl.BlockSpec((_NBLK, _WCOL), lambda i, c: (i, c)),
                   pl.BlockSpec((_NBLK, _WCOL), lambda i, c: (i, c)),
                   pl.BlockSpec((1, _NBLK, _WCOL), lambda i, c: (c, i, 0))],
        out_shape=[jax.ShapeDtypeStruct((n, c), _F32),
                   jax.ShapeDtypeStruct((n, c), _F32),
                   jax.ShapeDtypeStruct((nchunk, n, _WCOL), _F32)],
        input_output_aliases={3: 0},
    )(s3, deg, wbd, oa)


def _cheb_step2(s3, deg, wbd, txm2, oa, nchunk, n, c):
    def body(s3r, degr, wr, tmr, oir, oar, txr, y3r):
        dinv = _dinv_of(degr[...])
        tx = -2.0 * dinv * s3r[0] - tmr[...]
        txr[...] = tx
        oar[...] = oir[...] + jnp.dot(tx, wr[...], precision=_PREC)
        y3r[...] = (tx * dinv)[None]

    return pl.pallas_call(
        body,
        grid=(n // _NBLK, nchunk),
        in_specs=[pl.BlockSpec((1, _NBLK, _WCOL), lambda i, c: (c, i, 0)),
                  pl.BlockSpec((_NBLK, 1), lambda i, c: (i, 0)),
                  pl.BlockSpec((_WCOL, _WCOL), lambda i, c: (0, 0)),
                  pl.BlockSpec((_NBLK, _WCOL), lambda i, c: (i, c)),
                  pl.BlockSpec((_NBLK, _WCOL), lambda i, c: (i, c))],
        out_specs=[pl.BlockSpec((_NBLK, _WCOL), lambda i, c: (i, c)),
                   pl.BlockSpec((_NBLK, _WCOL), lambda i, c: (i, c)),
                   pl.BlockSpec((1, _NBLK, _WCOL), lambda i, c: (c, i, 0))],
        out_shape=[jax.ShapeDtypeStruct((n, c), _F32),
                   jax.ShapeDtypeStruct((n, c), _F32),
                   jax.ShapeDtypeStruct((nchunk, n, _WCOL), _F32)],
        input_output_aliases={4: 0},
    )(s3, deg, wbd, txm2, oa)


def _cheb_last(s3, deg, wbd, txm2, brow, oa, nchunk, n, c):
    def body(s3r, degr, wr, tmr, br, oir, outr):
        dinv = _dinv_of(degr[...])
        tx = -2.0 * dinv * s3r[0] - tmr[...]
        outr[...] = jax.nn.relu(
            oir[...] + jnp.dot(tx, wr[...], precision=_PREC) + br[...])

    return pl.pallas_call(
        body,
        grid=(n // _NBLK, nchunk),
        in_specs=[pl.BlockSpec((1, _NBLK, _WCOL), lambda i, c: (c, i, 0)),
                  pl.BlockSpec((_NBLK, 1), lambda i, c: (i, 0)),
                  pl.BlockSpec((_WCOL, _WCOL), lambda i, c: (0, 0)),
                  pl.BlockSpec((_NBLK, _WCOL), lambda i, c: (i, c)),
                  pl.BlockSpec((1, _WCOL), lambda i, c: (0, c)),
                  pl.BlockSpec((_NBLK, _WCOL), lambda i, c: (i, c))],
        out_specs=pl.BlockSpec((_NBLK, _WCOL), lambda i, c: (i, c)),
        out_shape=jax.ShapeDtypeStruct((n, c), _F32),
        input_output_aliases={5: 0},
    )(s3, deg, wbd, txm2, brow, oa)


def _bn_relu(t2, g, b):
    n, tt = t2.shape

    def body(tr, gr, br, outr):
        tb = tr[...]
        m = jnp.mean(tb, axis=1, keepdims=True)
        v = jnp.mean((tb - m) ** 2, axis=1, keepdims=True)
        outr[...] = jax.nn.relu((tb - m) / jnp.sqrt(v + 1e-5) * gr[...] + br[...])

    return pl.pallas_call(
        body,
        grid=(n // _NBLK,),
        in_specs=[pl.BlockSpec((_NBLK, tt), lambda i: (i, 0)),
                  pl.BlockSpec((_NBLK, 1), lambda i: (i, 0)),
                  pl.BlockSpec((_NBLK, 1), lambda i: (i, 0))],
        out_specs=pl.BlockSpec((_NBLK, tt), lambda i: (i, 0)),
        out_shape=jax.ShapeDtypeStruct((n, tt), _F32),
    )(t2, g, b)


def _final_linear(lw, h, lb):
    n = lw.shape[0]
    bb = h.shape[1]
    rows = 200

    def body(lwr, hr, lbr, outr):
        outr[...] = jnp.dot(lwr[...], hr[...], precision=_PREC) + lbr[...]

    return pl.pallas_call(
        body,
        grid=(n // rows,),
        in_specs=[pl.BlockSpec((rows, n), lambda i: (i, 0)),
                  pl.BlockSpec((n, bb), lambda i: (0, 0)),
                  pl.BlockSpec((rows, 1), lambda i: (i, 0))],
        out_specs=pl.BlockSpec((rows, bb), lambda i: (i, 0)),
        out_shape=jax.ShapeDtypeStruct((n, bb), _F32),
    )(lw, h, lb)


def _tconv_in_mat(w, t_in):
    t_out = t_in - (w.shape[-1] - 1)
    wr = w[:, 0, :].T
    es = jnp.stack([jnp.eye(t_out, t_in, k) for k in range(w.shape[-1])])
    mblk = jnp.einsum('ktn,ko->nto', es, wr, precision=_EXACT).reshape(t_in, t_out * _HID)
    return jnp.kron(jnp.eye(_B), mblk), t_out


def _tconv_out_mat(w, t_in):
    t_out = t_in - (w.shape[-1] - 1)
    wr = w[0].T
    es = jnp.stack([jnp.eye(t_out, t_in, k) for k in range(w.shape[-1])])
    mblk = jnp.einsum('ktn,kc->nct', es, wr, precision=_EXACT).reshape(t_in * _HID, t_out)
    return jnp.kron(jnp.eye(_B), mblk), t_out


def _bdiag(w, reps):
    return jnp.kron(jnp.eye(reps), w)


def _stconv(xin, src, dst, deg, t_in, p):
    m1, t1 = _tconv_in_mat(p['t1w1'], t_in)
    m2, _ = _tconv_in_mat(p['t1w2'], t_in)
    m3, _ = _tconv_in_mat(p['t1w3'], t_in)
    b1 = jnp.tile(p['t1b1'], _B * t1)[None]
    b2 = jnp.tile(p['t1b2'], _B * t1)[None]
    b3 = jnp.tile(p['t1b3'], _B * t1)[None]
    t0 = _gated(xin, m1, m2, m3, b1, b2, b3)

    cc = _B * t1 * _HID
    nchunk = cc // _WCOL
    assert cc % _WCOL == 0
    reps = _WCOL // _HID
    chw = p['chw']
    oa, y3 = _cheb_first(t0, deg, _bdiag(chw[0], reps), nchunk)
    y_list = [y3[c] for c in range(nchunk)]
    s3 = _sc_scatter_call(nchunk, _WCOL, _E, src, dst, y_list)
    oa, tx1, y3 = _cheb_step1(s3, deg, _bdiag(chw[1], reps), oa, nchunk, _N, cc)
    txm2 = t0
    txprev = tx1
    for k in range(2, chw.shape[0] - 1):
        y_list = [y3[c] for c in range(nchunk)]
        s3 = _sc_scatter_call(nchunk, _WCOL, _E, src, dst, y_list)
        oa, txk, y3 = _cheb_step2(s3, deg, _bdiag(chw[k], reps), txm2, oa,
                                  nchunk, _N, cc)
        txm2, txprev = txprev, txk
    y_list = [y3[c] for c in range(nchunk)]
    s3 = _sc_scatter_call(nchunk, _WCOL, _E, src, dst, y_list)
    brow = jnp.tile(p['chb'], _B * t1)[None]
    g = _cheb_last(s3, deg, _bdiag(chw[-1], reps), txm2, brow, oa,
                   nchunk, _N, cc)

    m1o, t2 = _tconv_out_mat(p['t2w1'], t1)
    m2o, _ = _tconv_out_mat(p['t2w2'], t1)
    m3o, _ = _tconv_out_mat(p['t2w3'], t1)
    b1o = jnp.tile(p['t2b1'], _B * t2)[None]
    b2o = jnp.tile(p['t2b2'], _B * t2)[None]
    b3o = jnp.tile(p['t2b3'], _B * t2)[None]
    t2a = _gated(g, m1o, m2o, m3o, b1o, b2o, b3o)

    return _bn_relu(t2a, p['bng'][:, None], p['bnb'][:, None])


def kernel(x, edge_index, a_t1w1, a_t1b1, a_t2w1, a_t2b1, a_t1w2, a_t1b2,
           a_t2w2, a_t2b2, a_t1w3, a_t1b3, a_t2w3, a_t2b3, a_chw, a_chb,
           a_bng, a_bnb, b_t1w1, b_t1b1, b_t2w1, b_t2b1, b_t1w2, b_t1b2,
           b_t2w2, b_t2b2, b_t1w3, b_t1b3, b_t2w3, b_t2b3, b_chw, b_chb,
           b_bng, b_bnb, lin_w, lin_b):
    src = edge_index[0]
    dst = edge_index[1]

    x_nm = jnp.transpose(x[:, :, :, 0], (2, 0, 1)).reshape(_N, -1)

    deg = _sc_deg_call(src)

    pa = {'t1w1': a_t1w1, 't1b1': a_t1b1, 't1w2': a_t1w2, 't1b2': a_t1b2,
          't1w3': a_t1w3, 't1b3': a_t1b3, 't2w1': a_t2w1, 't2b1': a_t2b1,
          't2w2': a_t2w2, 't2b2': a_t2b2, 't2w3': a_t2w3, 't2b3': a_t2b3,
          'chw': a_chw, 'chb': a_chb, 'bng': a_bng, 'bnb': a_bnb}
    pb = {'t1w1': b_t1w1, 't1b1': b_t1b1, 't1w2': b_t1w2, 't1b2': b_t1b2,
          't1w3': b_t1w3, 't1b3': b_t1b3, 't2w1': b_t2w1, 't2b1': b_t2b1,
          't2w2': b_t2w2, 't2b2': b_t2b2, 't2w3': b_t2w3, 't2b3': b_t2b3,
          'chw': b_chw, 'chb': b_chb, 'bng': b_bng, 'bnb': b_bnb}

    h = _stconv(x_nm, src, dst, deg, 12, pa)
    h = _stconv(h, src, dst, deg, 8, pb)

    hsel = h.reshape(_N, _B, 4)[:, :, 3]
    out_nm = _final_linear(lin_w, hsel, lin_b[:, None])
    return out_nm.T


# speedup vs baseline: 1.8287x; 1.6337x over previous
import functools

import jax
import jax.numpy as jnp
from jax import lax
from jax.experimental import pallas as pl
from jax.experimental.pallas import tpu as pltpu
from jax.experimental.pallas import tpu_sc as plsc

_N = 10000
_E = 160000
_HID = 16
_B = 8
_WCOL = 128
_NBLK = 400
_SC_NC = 2
_SC_NS = 16
_GE = 80
_QB = 32
_STRIPE = _N // _SC_NS
_PREC = jax.lax.Precision.DEFAULT
_EXACT = jax.lax.Precision.HIGHEST
_F32 = jnp.float32


@functools.lru_cache(maxsize=None)
def _make_sc_scatter(nchunk, width, n_edges):
    ep = n_edges // _SC_NS
    nit = ep // _GE
    assert ep % _GE == 0 and ep % 8 == 0 and nit >= 2
    mesh = plsc.VectorSubcoreMesh(core_axis_name="c", subcore_axis_name="s")
    out_type = jax.ShapeDtypeStruct((nchunk, _N, width), _F32)
    nq = -(-nit // _QB)
    nit_pad = nq * _QB
    scratch = [
        pltpu.VMEM((_QB, _GE), jnp.int32),
        pltpu.VMEM((_QB, _GE), jnp.int32),
        pltpu.VMEM((_GE, width), _F32),
        pltpu.VMEM((_GE, width), _F32),
        pltpu.VMEM_SHARED((_N, width), _F32),
        pltpu.SemaphoreType.DMA,
        pltpu.SemaphoreType.DMA,
        pltpu.SemaphoreType.DMA,
        pltpu.SemaphoreType.DMA,
    ]

    @functools.partial(pl.kernel, out_type=out_type, mesh=mesh,
                       scratch_types=scratch)
    def sck(src_hbm, dst_hbm, zeros_hbm, *rest):
        y_refs = rest[:nchunk]
        s_ref = rest[nchunk]
        (sidx, didx, rows_a, rows_b, acc,
         gsem_a, gsem_b, ssem_a, ssem_b) = rest[nchunk + 1:]
        cid = lax.axis_index("c")
        sid = lax.axis_index("s")
        s0 = 640
        last = _N - (_SC_NS - 1) * s0
        r0 = pl.multiple_of(sid * s0, 8)

        def _stripe(fn):
            @pl.when(sid < _SC_NS - 1)
            def _():
                fn(r0, s0)

            @pl.when(sid == _SC_NS - 1)
            def _():
                fn((_SC_NS - 1) * s0, last)

        def _buf(p):
            return ((rows_a, gsem_a, ssem_a), (rows_b, gsem_b, ssem_b))[p]

        def _run_quarter(yr, q, nb):
            pltpu.sync_copy(src_hbm.at[sid, pl.ds(q * _QB, _QB)], sidx)
            pltpu.sync_copy(dst_hbm.at[sid, pl.ds(q * _QB, _QB)], didx)
            pltpu.async_copy(yr.at[sidx.at[0]], rows_a, gsem_a)

            def body(g, carry):
                def step(cur, nxt):
                    crows, cgsem, cssem = cur
                    nrows, ngsem, nssem = nxt
                    pltpu.make_async_copy(
                        yr.at[sidx.at[g]], crows, cgsem).wait()

                    @pl.when(g + 1 < nb)
                    def _():
                        @pl.when(g >= 1)
                        def _():
                            pltpu.make_async_copy(
                                nrows, acc.at[didx.at[g]], nssem).wait()
                        pltpu.async_copy(
                            yr.at[sidx.at[g + 1]], nrows, ngsem)

                    pltpu.async_copy(
                        crows, acc.at[didx.at[g]], cssem, add=True)

                @pl.when(g % 2 == 0)
                def _():
                    step(_buf(0), _buf(1))

                @pl.when(g % 2 == 1)
                def _():
                    step(_buf(1), _buf(0))

                return carry

            lax.fori_loop(0, nb, body, 0)
            rl, _, sl = _buf((nb - 1) % 2)
            rp, _, sp = _buf(nb % 2)
            pltpu.make_async_copy(rl, acc.at[didx.at[0]], sl).wait()
            pltpu.make_async_copy(rp, acc.at[didx.at[0]], sp).wait()

        for c in range(nchunk):
            @pl.when(cid == (c % _SC_NC))
            def _(yr=y_refs[c], c=c):
                _stripe(lambda a, m: pltpu.sync_copy(
                    zeros_hbm.at[pl.ds(a, m)], acc.at[pl.ds(a, m)]))
                plsc.subcore_barrier()
                for q in range(nq):
                    _run_quarter(yr, q, min(_QB, nit - q * _QB))
                plsc.subcore_barrier()
                _stripe(lambda a, m: pltpu.sync_copy(
                    acc.at[pl.ds(a, m)], s_ref.at[c, pl.ds(a, m)]))
        return None

    return sck


def _sc_scatter_call(nchunk, width, n_edges, src, dst, y_list):
    ep = n_edges // _SC_NS
    nit = ep // _GE
    pad = -(-nit // _QB) * _QB - nit
    src3 = src.reshape(_SC_NS, nit, _GE)
    dst3 = dst.reshape(_SC_NS, nit, _GE)
    if pad:
        src3 = jnp.pad(src3, ((0, 0), (0, pad), (0, 0)))
        dst3 = jnp.pad(dst3, ((0, 0), (0, pad), (0, 0)))
    zeros = jnp.zeros((_N, width), _F32)
    fn = _make_sc_scatter(nchunk, width, n_edges)
    return fn(src3, dst3, zeros, *y_list)


@functools.lru_cache(maxsize=None)
def _make_sc_deg(nit):
    assert nit % _QB == 0
    nq = nit // _QB
    mesh = plsc.VectorSubcoreMesh(core_axis_name="c", subcore_axis_name="s")
    out_type = jax.ShapeDtypeStruct((_SC_NC, _N, _WCOL), _F32)
    scratch = [
        pltpu.VMEM((_QB, _GE), jnp.int32),
        pltpu.VMEM((_GE, _WCOL), _F32),
        pltpu.VMEM_SHARED((_N + 8, _WCOL), _F32),
        pltpu.SemaphoreType.DMA,
        pltpu.SemaphoreType.DMA,
    ]

    @functools.partial(pl.kernel, out_type=out_type, mesh=mesh,
                       scratch_types=scratch)
    def dk(dst_hbm, zeros_hbm, ones_hbm, s_ref, didx, ones_v, acc,
           ssem_a, ssem_b):
        cid = lax.axis_index("c")
        sid = lax.axis_index("s")
        w = cid * _SC_NS + sid
        s0 = 640
        last = _N - (_SC_NS - 1) * s0
        r0 = pl.multiple_of(sid * s0, 8)

        def _stripe(fn):
            @pl.when(sid < _SC_NS - 1)
            def _():
                fn(r0, s0)

            @pl.when(sid == _SC_NS - 1)
            def _():
                fn((_SC_NS - 1) * s0, last)

        def _scat(b, sem):
            return pltpu.make_async_copy(ones_v, acc.at[didx.at[b]], sem)

        pltpu.sync_copy(ones_hbm, ones_v)
        _stripe(lambda a, m: pltpu.sync_copy(
            zeros_hbm.at[pl.ds(a, m)], acc.at[pl.ds(a, m)]))
        plsc.subcore_barrier()
        for q in range(nq):
            pltpu.sync_copy(dst_hbm.at[w, pl.ds(q * _QB, _QB)], didx)

            def body(t, carry):
                b0 = 2 * t

                @pl.when(t >= 1)
                def _():
                    _scat(b0, ssem_a).wait()
                    _scat(b0, ssem_b).wait()

                _scat(b0, ssem_a).start(add=True)
                _scat(b0 + 1, ssem_b).start(add=True)
                return carry

            lax.fori_loop(0, _QB // 2, body, 0)
            _scat(0, ssem_a).wait()
            _scat(0, ssem_b).wait()
        plsc.subcore_barrier()
        _stripe(lambda a, m: pltpu.sync_copy(
            acc.at[pl.ds(a, m)], s_ref.at[cid, pl.ds(a, m)]))
        return None

    return dk


def _sc_deg_call(src):
    nw = _SC_NC * _SC_NS
    per = src.shape[0] // nw
    nb = -(-per // _GE)
    nit = -(-nb // _QB) * _QB
    padn = nit * _GE - per
    x = src.reshape(nw, per)
    if padn:
        pad = _N + (jnp.zeros((nw, padn), jnp.int32)
                    + jnp.arange(padn, dtype=jnp.int32) % 8)
        x = jnp.concatenate([x, pad], axis=1)
    dst3 = x.reshape(nw, nit, _GE)
    zeros = jnp.zeros((_N, _WCOL), _F32)
    ones = jnp.ones((_GE, _WCOL), _F32)
    d3 = _make_sc_deg(nit)(dst3, zeros, ones)
    return (d3[0, :, 0] + d3[1, :, 0])[:, None]


def _gated(x, m1, m2, m3, b1, b2, b3):
    n, cin = x.shape
    cout = m1.shape[1]

    def body(xr, m1r, m2r, m3r, b1r, b2r, b3r, outr):
        xb = xr[...]
        p = jnp.dot(xb, m1r[...], precision=_PREC) + b1r[...]
        q = jax.nn.sigmoid(jnp.dot(xb, m2r[...], precision=_PREC) + b2r[...])
        r = jnp.dot(xb, m3r[...], precision=_PREC) + b3r[...]
        outr[...] = jax.nn.relu(p * q + r)

    wspec = pl.BlockSpec((cin, cout), lambda i: (0, 0))
    bspec = pl.BlockSpec((1, cout), lambda i: (0, 0))
    return pl.pallas_call(
        body,
        grid=(n // _NBLK,),
        in_specs=[pl.BlockSpec((_NBLK, cin), lambda i: (i, 0)),
                  wspec, wspec, wspec, bspec, bspec, bspec],
        out_specs=pl.BlockSpec((_NBLK, cout), lambda i: (i, 0)),
        out_shape=jax.ShapeDtypeStruct((n, cout), _F32),
    )(x, m1, m2, m3, b1, b2, b3)


def _dinv_of(d):
    return jnp.where(d > 0, 1.0 / jnp.sqrt(jnp.maximum(d, 1.0)), 0.0)


def _cheb_first(t0, deg, w0bd, nchunk):
    n, c = t0.shape

    def body(t0r, degr, w0r, oar, y3r):
        dinv = _dinv_of(degr[...])
        tb = t0r[...]
        oar[...] = jnp.dot(tb, w0r[...], precision=_PREC)
        y3r[...] = (tb * dinv)[None]

    return pl.pallas_call(
        body,
        grid=(n // _NBLK, nchunk),
        in_specs=[pl.BlockSpec((_NBLK, _WCOL), lambda i, c: (i, c)),
                  pl.BlockSpec((_NBLK, 1), lambda i, c: (i, 0)),
                  pl.BlockSpec((_WCOL, _WCOL), lambda i, c: (0, 0))],
        out_specs=[pl.BlockSpec((_NBLK, _WCOL), lambda i, c: (i, c)),
                   pl.BlockSpec((1, _NBLK, _WCOL), lambda i, c: (c, i, 0))],
        out_shape=[jax.ShapeDtypeStruct((n, c), _F32),
                   jax.ShapeDtypeStruct((nchunk, n, _WCOL), _F32)],
    )(t0, deg, w0bd)


def _cheb_step1(s3, deg, wbd, oa, nchunk, n, c):
    def body(s3r, degr, wr, oir, oar, txr, y3r):
        dinv = _dinv_of(degr[...])
        tx = -dinv * s3r[0]
        txr[...] = tx
        oar[...] = oir[...] + jnp.dot(tx, wr[...], precision=_PREC)
        y3r[...] = (tx * dinv)[None]

    return pl.pallas_call(
        body,
        grid=(n // _NBLK, nchunk),
        in_specs=[pl.BlockSpec((1, _NBLK, _WCOL), lambda i, c: (c, i, 0)),
                  pl.BlockSpec((_NBLK, 1), lambda i, c: (i, 0)),
                  pl.BlockSpec((_WCOL, _WCOL), lambda i, c: (0, 0)),
                  pl.BlockSpec((_NBLK, _WCOL), lambda i, c: (i, c))],
        out_specs=[pl.BlockSpec((_NBLK, _WCOL), lambda i, c: (i, c)),
                   pl.BlockSpec((_NBLK, _WCOL), lambda i, c: (i, c)),
                   pl.BlockSpec((1, _NBLK, _WCOL), lambda i, c: (c, i, 0))],
        out_shape=[jax.ShapeDtypeStruct((n, c), _F32),
                   jax.ShapeDtypeStruct((n, c), _F32),
                   jax.ShapeDtypeStruct((nchunk, n, _WCOL), _F32)],
        input_output_aliases={3: 0},
    )(s3, deg, wbd, oa)


def _cheb_step2(s3, deg, wbd, txm2, oa, nchunk, n, c):
    def body(s3r, degr, wr, tmr, oir, oar, txr, y3r):
        dinv = _dinv_of(degr[...])
        tx = -2.0 * dinv * s3r[0] - tmr[...]
        txr[...] = tx
        oar[...] = oir[...] + jnp.dot(tx, wr[...], precision=_PREC)
        y3r[...] = (tx * dinv)[None]

    return pl.pallas_call(
        body,
        grid=(n // _NBLK, nchunk),
        in_specs=[pl.BlockSpec((1, _NBLK, _WCOL), lambda i, c: (c, i, 0)),
                  pl.BlockSpec((_NBLK, 1), lambda i, c: (i, 0)),
                  pl.BlockSpec((_WCOL, _WCOL), lambda i, c: (0, 0)),
                  pl.BlockSpec((_NBLK, _WCOL), lambda i, c: (i, c)),
                  pl.BlockSpec((_NBLK, _WCOL), lambda i, c: (i, c))],
        out_specs=[pl.BlockSpec((_NBLK, _WCOL), lambda i, c: (i, c)),
                   pl.BlockSpec((_NBLK, _WCOL), lambda i, c: (i, c)),
                   pl.BlockSpec((1, _NBLK, _WCOL), lambda i, c: (c, i, 0))],
        out_shape=[jax.ShapeDtypeStruct((n, c), _F32),
                   jax.ShapeDtypeStruct((n, c), _F32),
                   jax.ShapeDtypeStruct((nchunk, n, _WCOL), _F32)],
        input_output_aliases={4: 0},
    )(s3, deg, wbd, txm2, oa)


def _cheb_last(s3, deg, wbd, txm2, brow, oa, nchunk, n, c):
    def body(s3r, degr, wr, tmr, br, oir, outr):
        dinv = _dinv_of(degr[...])
        tx = -2.0 * dinv * s3r[0] - tmr[...]
        outr[...] = jax.nn.relu(
            oir[...] + jnp.dot(tx, wr[...], precision=_PREC) + br[...])

    return pl.pallas_call(
        body,
        grid=(n // _NBLK, nchunk),
        in_specs=[pl.BlockSpec((1, _NBLK, _WCOL), lambda i, c: (c, i, 0)),
                  pl.BlockSpec((_NBLK, 1), lambda i, c: (i, 0)),
                  pl.BlockSpec((_WCOL, _WCOL), lambda i, c: (0, 0)),
                  pl.BlockSpec((_NBLK, _WCOL), lambda i, c: (i, c)),
                  pl.BlockSpec((1, _WCOL), lambda i, c: (0, c)),
                  pl.BlockSpec((_NBLK, _WCOL), lambda i, c: (i, c))],
        out_specs=pl.BlockSpec((_NBLK, _WCOL), lambda i, c: (i, c)),
        out_shape=jax.ShapeDtypeStruct((n, c), _F32),
        input_output_aliases={5: 0},
    )(s3, deg, wbd, txm2, brow, oa)


def _bn_relu(t2, g, b):
    n, tt = t2.shape

    def body(tr, gr, br, outr):
        tb = tr[...]
        m = jnp.mean(tb, axis=1, keepdims=True)
        v = jnp.mean((tb - m) ** 2, axis=1, keepdims=True)
        outr[...] = jax.nn.relu((tb - m) / jnp.sqrt(v + 1e-5) * gr[...] + br[...])

    return pl.pallas_call(
        body,
        grid=(n // _NBLK,),
        in_specs=[pl.BlockSpec((_NBLK, tt), lambda i: (i, 0)),
                  pl.BlockSpec((_NBLK, 1), lambda i: (i, 0)),
                  pl.BlockSpec((_NBLK, 1), lambda i: (i, 0))],
        out_specs=pl.BlockSpec((_NBLK, tt), lambda i: (i, 0)),
        out_shape=jax.ShapeDtypeStruct((n, tt), _F32),
    )(t2, g, b)


def _final_linear(lw, h, lb):
    n = lw.shape[0]
    bb = h.shape[1]
    rows = 200

    def body(lwr, hr, lbr, outr):
        outr[...] = jnp.dot(lwr[...], hr[...], precision=_PREC) + lbr[...]

    return pl.pallas_call(
        body,
        grid=(n // rows,),
        in_specs=[pl.BlockSpec((rows, n), lambda i: (i, 0)),
                  pl.BlockSpec((n, bb), lambda i: (0, 0)),
                  pl.BlockSpec((rows, 1), lambda i: (i, 0))],
        out_specs=pl.BlockSpec((rows, bb), lambda i: (i, 0)),
        out_shape=jax.ShapeDtypeStruct((n, bb), _F32),
    )(lw, h, lb)


def _tconv_in_mat(w, t_in):
    t_out = t_in - (w.shape[-1] - 1)
    wr = w[:, 0, :].T
    es = jnp.stack([jnp.eye(t_out, t_in, k) for k in range(w.shape[-1])])
    mblk = jnp.einsum('ktn,ko->nto', es, wr, precision=_EXACT).reshape(t_in, t_out * _HID)
    return jnp.kron(jnp.eye(_B), mblk), t_out


def _tconv_out_mat(w, t_in):
    t_out = t_in - (w.shape[-1] - 1)
    wr = w[0].T
    es = jnp.stack([jnp.eye(t_out, t_in, k) for k in range(w.shape[-1])])
    mblk = jnp.einsum('ktn,kc->nct', es, wr, precision=_EXACT).reshape(t_in * _HID, t_out)
    return jnp.kron(jnp.eye(_B), mblk), t_out


def _bdiag(w, reps):
    return jnp.kron(jnp.eye(reps), w)


def _stconv(xin, src, dst, deg, t_in, p):
    m1, t1 = _tconv_in_mat(p['t1w1'], t_in)
    m2, _ = _tconv_in_mat(p['t1w2'], t_in)
    m3, _ = _tconv_in_mat(p['t1w3'], t_in)
    b1 = jnp.tile(p['t1b1'], _B * t1)[None]
    b2 = jnp.tile(p['t1b2'], _B * t1)[None]
    b3 = jnp.tile(p['t1b3'], _B * t1)[None]
    t0 = _gated(xin, m1, m2, m3, b1, b2, b3)

    cc = _B * t1 * _HID
    nchunk = cc // _WCOL
    assert cc % _WCOL == 0
    reps = _WCOL // _HID
    chw = p['chw']
    oa, y3 = _cheb_first(t0, deg, _bdiag(chw[0], reps), nchunk)
    y_list = [y3[c] for c in range(nchunk)]
    s3 = _sc_scatter_call(nchunk, _WCOL, _E, src, dst, y_list)
    oa, tx1, y3 = _cheb_step1(s3, deg, _bdiag(chw[1], reps), oa, nchunk, _N, cc)
    txm2 = t0
    txprev = tx1
    for k in range(2, chw.shape[0] - 1):
        y_list = [y3[c] for c in range(nchunk)]
        s3 = _sc_scatter_call(nchunk, _WCOL, _E, src, dst, y_list)
        oa, txk, y3 = _cheb_step2(s3, deg, _bdiag(chw[k], reps), txm2, oa,
                                  nchunk, _N, cc)
        txm2, txprev = txprev, txk
    y_list = [y3[c] for c in range(nchunk)]
    s3 = _sc_scatter_call(nchunk, _WCOL, _E, src, dst, y_list)
    brow = jnp.tile(p['chb'], _B * t1)[None]
    g = _cheb_last(s3, deg, _bdiag(chw[-1], reps), txm2, brow, oa,
                   nchunk, _N, cc)

    m1o, t2 = _tconv_out_mat(p['t2w1'], t1)
    m2o, _ = _tconv_out_mat(p['t2w2'], t1)
    m3o, _ = _tconv_out_mat(p['t2w3'], t1)
    b1o = jnp.tile(p['t2b1'], _B * t2)[None]
    b2o = jnp.tile(p['t2b2'], _B * t2)[None]
    b3o = jnp.tile(p['t2b3'], _B * t2)[None]
    t2a = _gated(g, m1o, m2o, m3o, b1o, b2o, b3o)

    return _bn_relu(t2a, p['bng'][:, None], p['bnb'][:, None])


def kernel(x, edge_index, a_t1w1, a_t1b1, a_t2w1, a_t2b1, a_t1w2, a_t1b2,
           a_t2w2, a_t2b2, a_t1w3, a_t1b3, a_t2w3, a_t2b3, a_chw, a_chb,
           a_bng, a_bnb, b_t1w1, b_t1b1, b_t2w1, b_t2b1, b_t1w2, b_t1b2,
           b_t2w2, b_t2b2, b_t1w3, b_t1b3, b_t2w3, b_t2b3, b_chw, b_chb,
           b_bng, b_bnb, lin_w, lin_b):
    src = edge_index[0]
    dst = edge_index[1]

    x_nm = jnp.transpose(x[:, :, :, 0], (2, 0, 1)).reshape(_N, -1)

    deg = _sc_deg_call(src)

    pa = {'t1w1': a_t1w1, 't1b1': a_t1b1, 't1w2': a_t1w2, 't1b2': a_t1b2,
          't1w3': a_t1w3, 't1b3': a_t1b3, 't2w1': a_t2w1, 't2b1': a_t2b1,
          't2w2': a_t2w2, 't2b2': a_t2b2, 't2w3': a_t2w3, 't2b3': a_t2b3,
          'chw': a_chw, 'chb': a_chb, 'bng': a_bng, 'bnb': a_bnb}
    pb = {'t1w1': b_t1w1, 't1b1': b_t1b1, 't1w2': b_t1w2, 't1b2': b_t1b2,
          't1w3': b_t1w3, 't1b3': b_t1b3, 't2w1': b_t2w1, 't2b1': b_t2b1,
          't2w2': b_t2w2, 't2b2': b_t2b2, 't2w3': b_t2w3, 't2b3': b_t2b3,
          'chw': b_chw, 'chb': b_chb, 'bng': b_bng, 'bnb': b_bnb}

    h = _stconv(x_nm, src, dst, deg, 12, pa)
    h = _stconv(h, src, dst, deg, 8, pb)

    hsel = h.reshape(_N, _B, 4)[:, :, 3]
    out_nm = _final_linear(lin_w, hsel, lin_b[:, None])
    return out_nm.T
